# baseline (speedup 1.0000x reference)
import jax
import jax.numpy as jnp
from jax.experimental import pallas as pl
from jax.experimental.pallas import tpu as pltpu

_BATCH_TILE = 64


def _fwd_kernel(tok_ref, tgt_ref, emb_ref, embh_ref, logits_ref, loss_ref):
    nb, t = tok_ref.shape
    v = emb_ref.shape[0]
    emb = emb_ref[...]
    r = nb * t
    tok_row = jnp.concatenate([tok_ref[b:b + 1, :] for b in range(nb)], axis=1)
    tgt_row = jnp.concatenate([tgt_ref[b:b + 1, :] for b in range(nb)], axis=1)
    row = jax.lax.broadcasted_iota(jnp.int32, (v, r), 0)

    oht_tok = (row == tok_row).astype(jnp.bfloat16)
    oht_tgt = (row == tgt_row).astype(jnp.bfloat16)
    logits_ref[...] = jax.lax.dot_general(
        oht_tok, embh_ref[...], (((0,), (0,)), ((), ())),
        preferred_element_type=jnp.float32)
    pc = jax.lax.dot_general(oht_tok, oht_tgt, (((1,), (1,)), ((), ())),
                             preferred_element_type=jnp.float32)

    m = jnp.max(emb, axis=-1, keepdims=True)
    lse_v = jnp.log(jnp.sum(jnp.exp(emb - m), axis=-1, keepdims=True)) + m
    cnt_tok = jnp.sum(pc, axis=-1, keepdims=True)
    block_loss = jnp.sum(cnt_tok * lse_v) - jnp.sum(pc * emb)
    loss_ref[...] = jnp.broadcast_to(block_loss, loss_ref.shape)


def kernel(tokens, targets, emb_table):
    b, t = tokens.shape
    v = emb_table.shape[0]
    n = b * t

    tok = tokens.astype(jnp.int32)
    tgt = targets.astype(jnp.int32)

    num_blocks = pl.cdiv(b, _BATCH_TILE)
    b_pad = num_blocks * _BATCH_TILE
    if b_pad != b:
        tok = jnp.pad(tok, ((0, b_pad - b), (0, 0)))
        tgt = jnp.pad(tgt, ((0, b_pad - b), (0, 0)), constant_values=-1)

    emb_bf16 = emb_table.astype(jnp.bfloat16)

    tile_spec = pl.BlockSpec((_BATCH_TILE, t), lambda i: (i, 0))
    table_spec = pl.BlockSpec((v, v), lambda i: (0, 0))
    logits, loss_parts = pl.pallas_call(
        _fwd_kernel,
        grid=(num_blocks,),
        in_specs=[tile_spec, tile_spec, table_spec, table_spec],
        out_specs=(
            pl.BlockSpec((_BATCH_TILE * t, v), lambda i: (i, 0)),
            pl.BlockSpec((1, 8, 128), lambda i: (i, 0, 0)),
        ),
        out_shape=(
            jax.ShapeDtypeStruct((b_pad * t, v), jnp.float32),
            jax.ShapeDtypeStruct((num_blocks, 8, 128), jnp.float32),
        ),
        compiler_params=pltpu.CompilerParams(
            dimension_semantics=("parallel",),
            vmem_limit_bytes=60 * 1024 * 1024,
        ),
        cost_estimate=pl.CostEstimate(
            flops=4 * b_pad * t * v * v,
            transcendentals=num_blocks * v * v,
            bytes_accessed=b_pad * t * v * 4 + 2 * b_pad * t * 4 + v * v * 4,
        ),
    )(tok, tgt, emb_table, emb_bf16)

    loss = jnp.sum(loss_parts[:, 0, 0]) / jnp.float32(n)
    logits = logits[:n] if b_pad != b else logits
    return logits, loss
